# Initial kernel scaffold; baseline (speedup 1.0000x reference)
import jax
import jax.numpy as jnp
from jax.experimental import pallas as pl


def kernel(x, edge_index, batch, W1, att_src1, att_dst1, b1, g1, be1, W2, att_src2, att_dst2, b2, g2, be2, Wl, bl):
    raise NotImplementedError("write your pallas kernel here")


# speedup vs baseline: 68.5720x; 68.5720x over previous
import functools

import jax
import jax.numpy as jnp
import numpy as np
from jax import lax
from jax.experimental import pallas as pl
from jax.experimental.pallas import tpu as pltpu
from jax.experimental.pallas import tpu_sc as plsc

N = 10000
E = 320000
FEAT = 128
H = 16
C = 8
NC = 10
G = 64
HC = H * C
TBL = HC + H
CHUNK = 128
NROWS = E // CHUNK
NWORK = 32
ROWS_PW = NROWS // NWORK
ROWS_EXTRA = NROWS - ROWS_PW * NWORK
NPAD = 10112
NPT = NPAD // 16

_PERM = np.arange(HC).reshape(H, C).T.reshape(-1)

f32 = jnp.float32


def _dense1_body(x_ref, w_ref, as_ref, ad_ref, hs_out, ad_out):
    h = jnp.dot(x_ref[...], w_ref[...], preferred_element_type=f32)
    a_s = jnp.dot(h, as_ref[...], preferred_element_type=f32)
    hs_out[...] = jnp.concatenate([h, a_s], axis=1)
    ad_out[...] = jnp.dot(h, ad_ref[...], preferred_element_type=f32)


def _norm1_body(pp_ref, b_ref, o_ref):
    t = pp_ref[0] + pp_ref[1]
    num = t[:, :HC]
    r = 1.0 / (t[:, HC:] + 1e-16)
    rexp = jnp.concatenate([r] * C, axis=1)
    o = num * rexp + b_ref[...]
    o_ref[...] = jnp.where(o > 0, o, jnp.exp(o) - 1.0)


def _stats_body(o_ref, mu_ref, var_ref):
    o = o_ref[:N]
    mu = jnp.mean(o, axis=0, keepdims=True)
    mu_ref[...] = mu
    var_ref[...] = jnp.mean((o - mu) ** 2, axis=0, keepdims=True)


def _bnmm1_body(o_ref, mu_ref, var_ref, g_ref, be_ref, w2_ref, as2_ref,
                ad2_ref, hs_out, ad_out):
    ob = (o_ref[...] - mu_ref[...]) / jnp.sqrt(var_ref[...] + 1e-5) \
        * g_ref[...] + be_ref[...]
    h2 = jnp.dot(ob, w2_ref[...], preferred_element_type=f32)
    a_s = jnp.dot(h2, as2_ref[...], preferred_element_type=f32)
    hs_out[...] = jnp.concatenate([h2, a_s], axis=1)
    ad_out[...] = jnp.dot(h2, ad2_ref[...], preferred_element_type=f32)


def _norm2_body(pp_ref, mp_ref, b2_ref, o_ref):
    t = pp_ref[0] + pp_ref[1]
    num = t[:, :HC]
    r = 1.0 / (t[:, HC:] + 1e-16)
    rexp = jnp.concatenate([r] * C, axis=1)
    o_ref[...] = jnp.dot(num * rexp, mp_ref[...],
                         preferred_element_type=f32) + b2_ref[...]


def _final2_body(o_ref, g2_ref, be2_ref, batch_ref, wl_ref, bl_ref, out_ref):
    o = o_ref[:N]
    mu = jnp.mean(o, axis=0, keepdims=True)
    var = jnp.mean((o - mu) ** 2, axis=0, keepdims=True)
    ob = (o - mu) / jnp.sqrt(var + 1e-5) * g2_ref[...] + be2_ref[...]
    iot = lax.broadcasted_iota(jnp.int32, (G, N), 0)
    oh = (iot == batch_ref[...]).astype(f32)
    cnt = jnp.sum(oh, axis=1, keepdims=True)
    pooled = jnp.dot(oh, ob, preferred_element_type=f32)
    pooled = pooled / jnp.maximum(cnt, 1.0)
    out_ref[...] = jnp.dot(pooled, wl_ref[...], preferred_element_type=f32) \
        + bl_ref[...]


def _edge_body(hs_hbm, ad_hbm, src_hbm, dst_hbm, zero_hbm, out_hbm,
               src_v, dst_v, hs_rows, ad_rows, out_buf, acc):
    cid = lax.axis_index("c")
    sid = lax.axis_index("s")
    wid = cid * 16 + sid

    pltpu.sync_copy(zero_hbm.at[pl.ds(sid * NPT, NPT)],
                    acc.at[pl.ds(sid * NPT, NPT)])
    plsc.subcore_barrier()

    def do_chunk(row):
        pltpu.sync_copy(src_hbm.at[row], src_v)
        pltpu.sync_copy(dst_hbm.at[pl.ds(row, 1)], dst_v)
        pltpu.sync_copy(hs_hbm.at[src_v], hs_rows)
        pltpu.sync_copy(ad_hbm.at[dst_v.at[0]], ad_rows)

        @pl.loop(0, CHUNK)
        def _edge(i):
            a = hs_rows[i, pl.ds(HC, H)] + ad_rows[i, :]
            a = jnp.where(a > 0, a, 0.2 * a)
            p = jnp.exp(a)
            out_buf[i, pl.ds(HC, H)] = p
            for j in range(C):
                out_buf[i, pl.ds(16 * j, 16)] = hs_rows[i, pl.ds(16 * j, 16)] * p

        pltpu.sync_copy(out_buf, acc.at[dst_v.at[0]], add=True)

    @pl.loop(0, ROWS_PW)
    def _rows(k):
        do_chunk(wid * ROWS_PW + k)

    @pl.when(wid < ROWS_EXTRA)
    def _extra():
        do_chunk(NWORK * ROWS_PW + wid)

    plsc.subcore_barrier()
    pltpu.sync_copy(acc.at[pl.ds(sid * NPT, NPT)],
                    out_hbm.at[cid, pl.ds(sid * NPT, NPT)])


_edge_kernel = functools.partial(
    pl.kernel,
    out_type=jax.ShapeDtypeStruct((2, NPAD, TBL), f32),
    mesh=plsc.VectorSubcoreMesh(core_axis_name="c", subcore_axis_name="s"),
    scratch_types=[
        pltpu.VMEM((CHUNK,), jnp.int32),
        pltpu.VMEM((1, CHUNK), jnp.int32),
        pltpu.VMEM((CHUNK, TBL), f32),
        pltpu.VMEM((CHUNK, H), f32),
        pltpu.VMEM((CHUNK, TBL), f32),
        pltpu.VMEM_SHARED((NPAD, TBL), f32),
    ],
    compiler_params=pltpu.CompilerParams(use_tc_tiling_on_sc=False),
)(_edge_body)


def kernel(x, edge_index, batch, W1, att_src1, att_dst1, b1, g1, be1,
           W2, att_src2, att_dst2, b2, g2, be2, Wl, bl):
    perm = jnp.asarray(_PERM)
    eye16 = jnp.eye(H, dtype=f32)

    def amat(att):
        return (att.T[:, :, None] * eye16[None, :, :]).reshape(HC, H)

    W1p = W1[:, perm]
    W2pp = W2[perm][:, perm]
    A1s, A1d = amat(att_src1), amat(att_dst1)
    A2s, A2d = amat(att_src2), amat(att_dst2)
    b1p = b1[perm].reshape(1, HC)
    g1p = g1[perm].reshape(1, HC)
    be1p = be1[perm].reshape(1, HC)
    Mp = jnp.broadcast_to(jnp.eye(C, dtype=f32)[:, None, :] / H,
                          (C, H, C)).reshape(HC, C)
    src2d = edge_index[0].astype(jnp.int32).reshape(NROWS, CHUNK)
    dst2d = edge_index[1].astype(jnp.int32).reshape(NROWS, CHUNK)
    zeros = jnp.zeros((NPAD, TBL), f32)
    batch2d = batch.astype(jnp.int32).reshape(1, N)

    hs1, ad1 = pl.pallas_call(
        _dense1_body,
        out_shape=(jax.ShapeDtypeStruct((N, TBL), f32),
                   jax.ShapeDtypeStruct((N, H), f32)),
    )(x, W1p, A1s, A1d)

    pp1 = _edge_kernel(hs1, ad1, src2d, dst2d, zeros)

    NB1 = 16
    BLK1 = NPAD // NB1
    o1 = pl.pallas_call(
        _norm1_body,
        grid=(NB1,),
        in_specs=[
            pl.BlockSpec((2, BLK1, TBL), lambda i: (0, i, 0)),
            pl.BlockSpec((1, HC), lambda i: (0, 0)),
        ],
        out_specs=pl.BlockSpec((BLK1, HC), lambda i: (i, 0)),
        out_shape=jax.ShapeDtypeStruct((NPAD, HC), f32),
    )(pp1, b1p)

    mu1, var1 = pl.pallas_call(
        _stats_body,
        out_shape=(jax.ShapeDtypeStruct((1, HC), f32),
                   jax.ShapeDtypeStruct((1, HC), f32)),
    )(o1)

    NB2 = 10
    BLK2 = N // NB2
    hs2, ad2 = pl.pallas_call(
        _bnmm1_body,
        grid=(NB2,),
        in_specs=[
            pl.BlockSpec((BLK2, HC), lambda i: (i, 0)),
            pl.BlockSpec((1, HC), lambda i: (0, 0)),
            pl.BlockSpec((1, HC), lambda i: (0, 0)),
            pl.BlockSpec((1, HC), lambda i: (0, 0)),
            pl.BlockSpec((1, HC), lambda i: (0, 0)),
            pl.BlockSpec((HC, HC), lambda i: (0, 0)),
            pl.BlockSpec((HC, H), lambda i: (0, 0)),
            pl.BlockSpec((HC, H), lambda i: (0, 0)),
        ],
        out_specs=(pl.BlockSpec((BLK2, TBL), lambda i: (i, 0)),
                   pl.BlockSpec((BLK2, H), lambda i: (i, 0))),
        out_shape=(jax.ShapeDtypeStruct((N, TBL), f32),
                   jax.ShapeDtypeStruct((N, H), f32)),
    )(o1[:N], mu1, var1, g1p, be1p, W2pp, A2s, A2d)

    pp2 = _edge_kernel(hs2, ad2, src2d, dst2d, zeros)

    o2 = pl.pallas_call(
        _norm2_body,
        grid=(NB1,),
        in_specs=[
            pl.BlockSpec((2, BLK1, TBL), lambda i: (0, i, 0)),
            pl.BlockSpec((HC, C), lambda i: (0, 0)),
            pl.BlockSpec((1, C), lambda i: (0, 0)),
        ],
        out_specs=pl.BlockSpec((BLK1, C), lambda i: (i, 0)),
        out_shape=jax.ShapeDtypeStruct((NPAD, C), f32),
    )(pp2, Mp, b2.reshape(1, C))

    out = pl.pallas_call(
        _final2_body,
        out_shape=jax.ShapeDtypeStruct((G, NC), f32),
    )(o2, g2.reshape(1, C), be2.reshape(1, C), batch2d, Wl, bl.reshape(1, NC))
    return out
